# baseline (speedup 1.0000x reference)
import functools

import jax
import jax.numpy as jnp
import numpy as np
from jax import lax
from jax.experimental import layout as jex_layout
from jax.experimental import pallas as pl
from jax.experimental.pallas import tpu as pltpu
from jax.experimental.pallas import tpu_sc as plsc

B = 4096
S = 50
N_TOK = B * S
VOCAB = 1000000
N_TURNS = 4
OUT_DIM = 128

NC = 2
NS = 16
NW = NC * NS
TOK_PER_W = N_TOK // NW
CHUNK = 128
N_CHUNKS = TOK_PER_W // CHUNK

PANELS = VOCAB // 128
MAIN = PANELS * 128
TAIL = VOCAB - MAIN
PACK_OFF = np.float32(21845.0)

PB = 128
PACK_GRID = -(-PANELS // PB)

_pack_m = np.kron(np.eye(PB, dtype=np.float32),
                  np.array([[1.0, 16.0, 256.0, 4096.0]], np.float32))


def _pack_body(m_ref, x_ref, out_ref):
    out_ref[...] = lax.dot_general(
        m_ref[...], x_ref[...], (((1,), (0,)), ((), ())),
        preferred_element_type=jnp.float32,
    ) + PACK_OFF


def _pack_main(panels):
    return pl.pallas_call(
        _pack_body,
        grid=(PACK_GRID,),
        in_specs=[
            pl.BlockSpec((PB, 4 * PB), lambda i: (0, 0)),
            pl.BlockSpec((4 * PB, 128), lambda i: (i, 0)),
        ],
        out_specs=pl.BlockSpec((PB, 128), lambda i: (i, 0)),
        out_shape=jax.ShapeDtypeStruct((PANELS, 128), jnp.float32),
    )(jnp.asarray(_pack_m), panels)


def _packed_table(turns):
    view3 = turns[:MAIN].reshape(PANELS, 128, N_TURNS).transpose(0, 2, 1)
    view3 = jex_layout.with_layout_constraint(
        view3, jex_layout.Layout((0, 1, 2), tiling=((4, 128),))
    )
    panels = view3.reshape(PANELS * N_TURNS, 128)
    packed_main = _pack_main(panels).reshape(MAIN)
    radix_w = jnp.array([1.0, 16.0, 256.0, 4096.0], jnp.float32)
    packed_tail = turns[MAIN:] @ radix_w + PACK_OFF
    return jnp.concatenate([packed_main, packed_tail])


_sc_mesh = plsc.VectorSubcoreMesh(core_axis_name="c", subcore_axis_name="s")


@functools.partial(
    pl.kernel,
    mesh=_sc_mesh,
    out_type=jax.ShapeDtypeStruct((N_TOK,), jnp.float32),
    scratch_types=[
        pltpu.VMEM((TOK_PER_W,), jnp.int32),
        pltpu.VMEM((TOK_PER_W,), jnp.float32),
        pltpu.SemaphoreType.DMA,
    ],
)
def _sc_gather(idx_hbm, packed_hbm, out_hbm, idx_v, val_v, sem):
    wid = lax.axis_index("s") * NC + lax.axis_index("c")
    base = wid * TOK_PER_W
    pltpu.sync_copy(idx_hbm.at[pl.ds(base, TOK_PER_W)], idx_v)
    copies = []
    for j in range(N_CHUNKS):
        copies.append(
            pltpu.async_copy(
                packed_hbm.at[idx_v.at[pl.ds(j * CHUNK, CHUNK)]],
                val_v.at[pl.ds(j * CHUNK, CHUNK)],
                sem,
            )
        )
    for cp in copies:
        cp.wait()
    pltpu.sync_copy(val_v, out_hbm.at[pl.ds(base, TOK_PER_W)])


TOK_BLK = 4096
GRID = N_TOK // TOK_BLK


def _tc_body(packed_ref, w_ref, out_ref):
    p = packed_ref[...].astype(jnp.int32)
    x0 = (p & 15) - 5
    x1 = ((p >> 4) & 15) - 5
    x2 = ((p >> 8) & 15) - 5
    x3 = (p >> 12) - 5
    x = jnp.concatenate([x0, x1, x2, x3], axis=0).astype(jnp.bfloat16)
    xx = x * x
    xxx = xx * x
    ones = jnp.ones((1, TOK_BLK), jnp.bfloat16)
    pw = jnp.concatenate([ones, x, xx, xxx], axis=0)
    out_ref[...] = lax.dot_general(
        pw, w_ref[...], (((0,), (0,)), ((), ())),
        preferred_element_type=jnp.float32,
    )


def _tc_dense(packed_plane, w13):
    return pl.pallas_call(
        _tc_body,
        grid=(GRID,),
        in_specs=[
            pl.BlockSpec((1, TOK_BLK), lambda i: (0, i)),
            pl.BlockSpec((3 * N_TURNS + 1, OUT_DIM), lambda i: (0, 0)),
        ],
        out_specs=pl.BlockSpec((TOK_BLK, OUT_DIM), lambda i: (i, 0)),
        out_shape=jax.ShapeDtypeStruct((N_TOK, OUT_DIM), jnp.float32),
    )(packed_plane, w13)


def kernel(token_ids, turns, poly_coeffs):
    idx1d = token_ids.T.reshape(N_TOK)
    packed = _packed_table(turns)
    plane = _sc_gather(idx1d, packed)
    w12 = poly_coeffs[:, 1:, :].transpose(1, 0, 2).reshape(3 * N_TURNS, OUT_DIM)
    bias = jnp.sum(poly_coeffs[:, 0, :], axis=0).reshape(1, OUT_DIM)
    w13 = jnp.concatenate([bias, w12], axis=0).astype(jnp.bfloat16)
    out2d = _tc_dense(plane.reshape(1, N_TOK), w13)
    return out2d.reshape(S, B, OUT_DIM).transpose(1, 0, 2)


# speedup vs baseline: 1.1729x; 1.1729x over previous
import functools

import jax
import jax.numpy as jnp
import numpy as np
from jax import lax
from jax.experimental import layout as jex_layout
from jax.experimental import pallas as pl
from jax.experimental.pallas import tpu as pltpu
from jax.experimental.pallas import tpu_sc as plsc

B = 4096
S = 50
N_TOK = B * S
VOCAB = 1000000
N_TURNS = 4
OUT_DIM = 128

NC = 2
NS = 16
NW = NC * NS
TOK_PER_W = N_TOK // NW
CHUNK = 128
N_CHUNKS = TOK_PER_W // CHUNK

PANELS = VOCAB // 128
MAIN = PANELS * 128
TAIL = VOCAB - MAIN
PACK_OFF = np.float32(21845.0)

PB = 512
PACK_GRID = -(-PANELS // PB)

_pack_m = np.kron(np.eye(PB, dtype=np.float32),
                  np.array([[1.0, 16.0, 256.0, 4096.0]], np.float32))


def _pack_body(m_ref, x_ref, out_ref):
    out_ref[...] = lax.dot_general(
        m_ref[...], x_ref[...].astype(jnp.bfloat16), (((1,), (0,)), ((), ())),
        preferred_element_type=jnp.float32,
    ) + PACK_OFF


def _pack_main(panels):
    return pl.pallas_call(
        _pack_body,
        grid=(PACK_GRID,),
        in_specs=[
            pl.BlockSpec((PB, 4 * PB), lambda i: (0, 0)),
            pl.BlockSpec((4 * PB, 128), lambda i: (i, 0)),
        ],
        out_specs=pl.BlockSpec((PB, 128), lambda i: (i, 0)),
        out_shape=jax.ShapeDtypeStruct((PANELS, 128), jnp.float32),
    )(jnp.asarray(_pack_m, jnp.bfloat16), panels)


def _packed_table(turns):
    view3 = turns[:MAIN].reshape(PANELS, 128, N_TURNS).transpose(0, 2, 1)
    view3 = jex_layout.with_layout_constraint(
        view3, jex_layout.Layout((0, 1, 2), tiling=((4, 128),))
    )
    panels = view3.reshape(PANELS * N_TURNS, 128)
    packed_main = _pack_main(panels).reshape(MAIN)
    radix_w = jnp.array([1.0, 16.0, 256.0, 4096.0], jnp.float32)
    packed_tail = turns[MAIN:] @ radix_w + PACK_OFF
    return jnp.concatenate([packed_main, packed_tail])


_sc_mesh = plsc.VectorSubcoreMesh(core_axis_name="c", subcore_axis_name="s")


@functools.partial(
    pl.kernel,
    mesh=_sc_mesh,
    out_type=jax.ShapeDtypeStruct((N_TOK,), jnp.float32),
    scratch_types=[
        pltpu.VMEM((TOK_PER_W,), jnp.int32),
        pltpu.VMEM((TOK_PER_W,), jnp.float32),
        pltpu.SemaphoreType.DMA,
    ],
)
def _sc_gather(idx_hbm, packed_hbm, out_hbm, idx_v, val_v, sem):
    wid = lax.axis_index("s") * NC + lax.axis_index("c")
    base = wid * TOK_PER_W
    pltpu.sync_copy(idx_hbm.at[pl.ds(base, TOK_PER_W)], idx_v)
    copies = []
    for j in range(N_CHUNKS):
        copies.append(
            pltpu.async_copy(
                packed_hbm.at[idx_v.at[pl.ds(j * CHUNK, CHUNK)]],
                val_v.at[pl.ds(j * CHUNK, CHUNK)],
                sem,
            )
        )
    for cp in copies:
        cp.wait()
    pltpu.sync_copy(val_v, out_hbm.at[pl.ds(base, TOK_PER_W)])


TOK_BLK = 4096
GRID = N_TOK // TOK_BLK


def _tc_body(packed_ref, w_ref, out_ref):
    p = packed_ref[...].astype(jnp.int32)
    x0 = (p & 15) - 5
    x1 = ((p >> 4) & 15) - 5
    x2 = ((p >> 8) & 15) - 5
    x3 = (p >> 12) - 5
    x = jnp.concatenate([x0, x1, x2, x3], axis=0).astype(jnp.bfloat16)
    xx = x * x
    xxx = xx * x
    ones = jnp.ones((1, TOK_BLK), jnp.bfloat16)
    pw = jnp.concatenate([ones, x, xx, xxx], axis=0)
    out_ref[...] = lax.dot_general(
        pw, w_ref[...], (((0,), (0,)), ((), ())),
        preferred_element_type=jnp.float32,
    )


def _tc_dense(packed_plane, w13):
    return pl.pallas_call(
        _tc_body,
        grid=(GRID,),
        in_specs=[
            pl.BlockSpec((1, TOK_BLK), lambda i: (0, i)),
            pl.BlockSpec((3 * N_TURNS + 1, OUT_DIM), lambda i: (0, 0)),
        ],
        out_specs=pl.BlockSpec((TOK_BLK, OUT_DIM), lambda i: (i, 0)),
        out_shape=jax.ShapeDtypeStruct((N_TOK, OUT_DIM), jnp.float32),
    )(packed_plane, w13)


def kernel(token_ids, turns, poly_coeffs):
    idx1d = token_ids.T.reshape(N_TOK)
    packed = _packed_table(turns)
    plane = _sc_gather(idx1d, packed)
    w12 = poly_coeffs[:, 1:, :].transpose(1, 0, 2).reshape(3 * N_TURNS, OUT_DIM)
    bias = jnp.sum(poly_coeffs[:, 0, :], axis=0).reshape(1, OUT_DIM)
    w13 = jnp.concatenate([bias, w12], axis=0).astype(jnp.bfloat16)
    out2d = _tc_dense(plane.reshape(1, N_TOK), w13)
    return out2d.reshape(S, B, OUT_DIM).transpose(1, 0, 2)


# speedup vs baseline: 1.3155x; 1.1215x over previous
import functools

import jax
import jax.numpy as jnp
import numpy as np
from jax import lax
from jax.experimental import layout as jex_layout
from jax.experimental import pallas as pl
from jax.experimental.pallas import tpu as pltpu
from jax.experimental.pallas import tpu_sc as plsc

B = 4096
S = 50
N_TOK = B * S
VOCAB = 1000000
N_TURNS = 4
OUT_DIM = 128

NC = 2
NS = 16
NW = NC * NS
TOK_PER_W = N_TOK // NW
CHUNK = 128
N_CHUNKS = TOK_PER_W // CHUNK

PANELS = VOCAB // 128
MAIN = PANELS * 128
TAIL = VOCAB - MAIN
PACK_OFF = np.float32(21845.0)

PB = 512
PACK_GRID = -(-PANELS // PB)

_pack_m = np.kron(np.eye(PB, dtype=np.float32),
                  np.array([[1.0, 16.0, 256.0, 4096.0]], np.float32))


def _pack_body(m_ref, x_ref, out_ref):
    out_ref[...] = lax.dot_general(
        m_ref[...], x_ref[...].astype(jnp.bfloat16), (((1,), (0,)), ((), ())),
        preferred_element_type=jnp.float32,
    ) + PACK_OFF


def _pack_main(panels):
    return pl.pallas_call(
        _pack_body,
        grid=(PACK_GRID,),
        in_specs=[
            pl.BlockSpec((PB, 4 * PB), lambda i: (0, 0)),
            pl.BlockSpec((4 * PB, 128), lambda i: (i, 0)),
        ],
        out_specs=pl.BlockSpec((PB, 128), lambda i: (i, 0)),
        out_shape=jax.ShapeDtypeStruct((PANELS, 128), jnp.float32),
    )(jnp.asarray(_pack_m, jnp.bfloat16), panels)


def _packed_table(turns):
    view3 = turns.T[:, :MAIN].reshape(N_TURNS, PANELS, 128).transpose(1, 0, 2)
    view3 = jex_layout.with_layout_constraint(
        view3, jex_layout.Layout((0, 1, 2), tiling=((4, 128),))
    )
    panels = view3.reshape(PANELS * N_TURNS, 128)
    packed_main = _pack_main(panels).reshape(MAIN)
    radix_w = jnp.array([1.0, 16.0, 256.0, 4096.0], jnp.float32)
    packed_tail = turns[MAIN:] @ radix_w + PACK_OFF
    return jnp.concatenate([packed_main, packed_tail])


_sc_mesh = plsc.VectorSubcoreMesh(core_axis_name="c", subcore_axis_name="s")


@functools.partial(
    pl.kernel,
    mesh=_sc_mesh,
    out_type=jax.ShapeDtypeStruct((N_TOK,), jnp.float32),
    scratch_types=[
        pltpu.VMEM((TOK_PER_W,), jnp.int32),
        pltpu.VMEM((TOK_PER_W,), jnp.float32),
        pltpu.SemaphoreType.DMA,
    ],
)
def _sc_gather(idx_hbm, packed_hbm, out_hbm, idx_v, val_v, sem):
    wid = lax.axis_index("s") * NC + lax.axis_index("c")
    base = wid * TOK_PER_W
    pltpu.sync_copy(idx_hbm.at[pl.ds(base, TOK_PER_W)], idx_v)
    copies = []
    for j in range(N_CHUNKS):
        copies.append(
            pltpu.async_copy(
                packed_hbm.at[idx_v.at[pl.ds(j * CHUNK, CHUNK)]],
                val_v.at[pl.ds(j * CHUNK, CHUNK)],
                sem,
            )
        )
    for cp in copies:
        cp.wait()
    pltpu.sync_copy(val_v, out_hbm.at[pl.ds(base, TOK_PER_W)])


TOK_BLK = 8192
GRID = N_TOK // TOK_BLK


def _tc_body(packed_ref, w_ref, out_ref):
    p = packed_ref[...].astype(jnp.int32)
    x0 = (p & 15) - 5
    x1 = ((p >> 4) & 15) - 5
    x2 = ((p >> 8) & 15) - 5
    x3 = (p >> 12) - 5
    x = jnp.concatenate([x0, x1, x2, x3], axis=0).astype(jnp.bfloat16)
    xx = x * x
    xxx = xx * x
    ones = jnp.ones((1, TOK_BLK), jnp.bfloat16)
    pw = jnp.concatenate([ones, x, xx, xxx], axis=0)
    out_ref[...] = lax.dot_general(
        pw, w_ref[...], (((0,), (0,)), ((), ())),
        preferred_element_type=jnp.float32,
    )


def _tc_dense(packed_plane, w13):
    return pl.pallas_call(
        _tc_body,
        grid=(GRID,),
        in_specs=[
            pl.BlockSpec((1, TOK_BLK), lambda i: (0, i)),
            pl.BlockSpec((3 * N_TURNS + 1, OUT_DIM), lambda i: (0, 0)),
        ],
        out_specs=pl.BlockSpec((TOK_BLK, OUT_DIM), lambda i: (i, 0)),
        out_shape=jax.ShapeDtypeStruct((N_TOK, OUT_DIM), jnp.float32),
    )(packed_plane, w13)


def kernel(token_ids, turns, poly_coeffs):
    idx1d = token_ids.T.reshape(N_TOK)
    packed = _packed_table(turns)
    plane = _sc_gather(idx1d, packed)
    w12 = poly_coeffs[:, 1:, :].transpose(1, 0, 2).reshape(3 * N_TURNS, OUT_DIM)
    bias = jnp.sum(poly_coeffs[:, 0, :], axis=0).reshape(1, OUT_DIM)
    w13 = jnp.concatenate([bias, w12], axis=0).astype(jnp.bfloat16)
    out2d = _tc_dense(plane.reshape(1, N_TOK), w13)
    return out2d.reshape(S, B, OUT_DIM).transpose(1, 0, 2)


# speedup vs baseline: 1.3421x; 1.0203x over previous
import functools

import jax
import jax.numpy as jnp
import numpy as np
from jax import lax
from jax.experimental import layout as jex_layout
from jax.experimental import pallas as pl
from jax.experimental.pallas import tpu as pltpu
from jax.experimental.pallas import tpu_sc as plsc

B = 4096
S = 50
N_TOK = B * S
VOCAB = 1000000
N_TURNS = 4
OUT_DIM = 128

NC = 2
NS = 16
NW = NC * NS
TOK_PER_W = N_TOK // NW
CHUNK = 128
N_CHUNKS = TOK_PER_W // CHUNK

PANELS = VOCAB // 128
MAIN = PANELS * 128
TAIL = VOCAB - MAIN
PACK_OFF = np.float32(21845.0)

PB = 512
PACK_GRID = -(-PANELS // PB)

_pack_m = np.kron(np.eye(PB, dtype=np.float32),
                  np.array([[1.0, 16.0, 256.0, 4096.0]], np.float32))


def _pack_body(m_ref, x_ref, out_ref):
    out_ref[...] = lax.dot_general(
        m_ref[...], x_ref[...].astype(jnp.bfloat16), (((1,), (0,)), ((), ())),
        preferred_element_type=jnp.float32,
    ) + PACK_OFF


def _pack_main(panels):
    return pl.pallas_call(
        _pack_body,
        grid=(PACK_GRID,),
        in_specs=[
            pl.BlockSpec((PB, 4 * PB), lambda i: (0, 0)),
            pl.BlockSpec((4 * PB, 128), lambda i: (i, 0)),
        ],
        out_specs=pl.BlockSpec((PB, 128), lambda i: (i, 0)),
        out_shape=jax.ShapeDtypeStruct((PANELS, 128), jnp.float32),
    )(jnp.asarray(_pack_m, jnp.bfloat16), panels)


def _packed_table(turns):
    view3 = turns.T[:, :MAIN].reshape(N_TURNS, PANELS, 128).transpose(1, 0, 2)
    view3 = jex_layout.with_layout_constraint(
        view3, jex_layout.Layout((0, 1, 2), tiling=((4, 128),))
    )
    panels = view3.reshape(PANELS * N_TURNS, 128)
    packed_main = _pack_main(panels).reshape(MAIN)
    radix_w = jnp.array([1.0, 16.0, 256.0, 4096.0], jnp.float32)
    packed_tail = turns[MAIN:] @ radix_w + PACK_OFF
    return jnp.concatenate([packed_main, packed_tail])


_sc_mesh = plsc.VectorSubcoreMesh(core_axis_name="c", subcore_axis_name="s")


@functools.partial(
    pl.kernel,
    mesh=_sc_mesh,
    out_type=jax.ShapeDtypeStruct((N_TOK,), jnp.float32),
    scratch_types=[
        pltpu.VMEM((TOK_PER_W,), jnp.int32),
        pltpu.VMEM((TOK_PER_W,), jnp.float32),
        pltpu.SemaphoreType.DMA,
    ],
)
def _sc_gather(idx_hbm, packed_hbm, out_hbm, idx_v, val_v, sem):
    wid = lax.axis_index("s") * NC + lax.axis_index("c")
    base = wid * TOK_PER_W
    pltpu.sync_copy(idx_hbm.at[pl.ds(base, TOK_PER_W)], idx_v)
    copies = []
    for j in range(N_CHUNKS):
        copies.append(
            pltpu.async_copy(
                packed_hbm.at[idx_v.at[pl.ds(j * CHUNK, CHUNK)]],
                val_v.at[pl.ds(j * CHUNK, CHUNK)],
                sem,
            )
        )
    for cp in copies:
        cp.wait()
    pltpu.sync_copy(val_v, out_hbm.at[pl.ds(base, TOK_PER_W)])


TOK_BLK = 25600
GRID = N_TOK // TOK_BLK


def _tc_body(packed_ref, w_ref, out_ref):
    p = packed_ref[...].astype(jnp.int32)
    x0 = (p & 15) - 5
    x1 = ((p >> 4) & 15) - 5
    x2 = ((p >> 8) & 15) - 5
    x3 = (p >> 12) - 5
    x = jnp.concatenate([x0, x1, x2, x3], axis=0).astype(jnp.bfloat16)
    xx = x * x
    xxx = xx * x
    ones = jnp.ones((1, TOK_BLK), jnp.bfloat16)
    pw = jnp.concatenate([ones, x, xx, xxx], axis=0)
    out_ref[...] = lax.dot_general(
        pw, w_ref[...], (((0,), (0,)), ((), ())),
        preferred_element_type=jnp.float32,
    )


def _tc_dense(packed_plane, w13):
    return pl.pallas_call(
        _tc_body,
        grid=(GRID,),
        in_specs=[
            pl.BlockSpec((1, TOK_BLK), lambda i: (0, i)),
            pl.BlockSpec((3 * N_TURNS + 1, OUT_DIM), lambda i: (0, 0)),
        ],
        out_specs=pl.BlockSpec((TOK_BLK, OUT_DIM), lambda i: (i, 0)),
        out_shape=jax.ShapeDtypeStruct((N_TOK, OUT_DIM), jnp.float32),
    )(packed_plane, w13)


def kernel(token_ids, turns, poly_coeffs):
    idx1d = token_ids.T.reshape(N_TOK)
    packed = _packed_table(turns)
    plane = _sc_gather(idx1d, packed)
    w12 = poly_coeffs[:, 1:, :].transpose(1, 0, 2).reshape(3 * N_TURNS, OUT_DIM)
    bias = jnp.sum(poly_coeffs[:, 0, :], axis=0).reshape(1, OUT_DIM)
    w13 = jnp.concatenate([bias, w12], axis=0).astype(jnp.bfloat16)
    out2d = _tc_dense(plane.reshape(1, N_TOK), w13)
    return out2d.reshape(S, B, OUT_DIM).transpose(1, 0, 2)


# speedup vs baseline: 1.3981x; 1.0417x over previous
import functools

import jax
import jax.numpy as jnp
import numpy as np
from jax import lax
from jax.experimental import layout as jex_layout
from jax.experimental import pallas as pl
from jax.experimental.pallas import tpu as pltpu
from jax.experimental.pallas import tpu_sc as plsc

B = 4096
S = 50
N_TOK = B * S
VOCAB = 1000000
N_TURNS = 4
OUT_DIM = 128

NC = 2
NS = 16
NW = NC * NS
TOK_PER_W = N_TOK // NW
CHUNK = 128
N_CHUNKS = TOK_PER_W // CHUNK

PANELS = VOCAB // 128
MAIN = PANELS * 128
TAIL = VOCAB - MAIN
PACK_OFF = np.float32(21845.0)

PB = 512
PACK_GRID = -(-PANELS // PB)
MB_SUB = 128

_pack_m = np.kron(np.eye(MB_SUB, dtype=np.float32),
                  np.array([[1.0, 16.0, 256.0, 4096.0]], np.float32))


def _pack_body(m_ref, x_ref, out_ref):
    m = m_ref[...]
    for k in range(PB // MB_SUB):
        xb = x_ref[pl.ds(k * 4 * MB_SUB, 4 * MB_SUB), :].astype(jnp.bfloat16)
        out_ref[pl.ds(k * MB_SUB, MB_SUB), :] = lax.dot_general(
            m, xb, (((1,), (0,)), ((), ())),
            preferred_element_type=jnp.float32,
        ) + PACK_OFF


def _pack_main(panels):
    return pl.pallas_call(
        _pack_body,
        grid=(PACK_GRID,),
        in_specs=[
            pl.BlockSpec((MB_SUB, 4 * MB_SUB), lambda i: (0, 0)),
            pl.BlockSpec((4 * PB, 128), lambda i: (i, 0)),
        ],
        out_specs=pl.BlockSpec((PB, 128), lambda i: (i, 0)),
        out_shape=jax.ShapeDtypeStruct((PANELS, 128), jnp.float32),
    )(jnp.asarray(_pack_m, jnp.bfloat16), panels)


def _packed_table(turns):
    view3 = turns.T[:, :MAIN].reshape(N_TURNS, PANELS, 128).transpose(1, 0, 2)
    view3 = jex_layout.with_layout_constraint(
        view3, jex_layout.Layout((0, 1, 2), tiling=((4, 128),))
    )
    panels = view3.reshape(PANELS * N_TURNS, 128)
    packed_main = _pack_main(panels).reshape(MAIN)
    radix_w = jnp.array([1.0, 16.0, 256.0, 4096.0], jnp.float32)
    packed_tail = turns[MAIN:] @ radix_w + PACK_OFF
    return jnp.concatenate([packed_main, packed_tail])


_sc_mesh = plsc.VectorSubcoreMesh(core_axis_name="c", subcore_axis_name="s")


@functools.partial(
    pl.kernel,
    mesh=_sc_mesh,
    out_type=jax.ShapeDtypeStruct((N_TOK,), jnp.float32),
    scratch_types=[
        pltpu.VMEM((TOK_PER_W,), jnp.int32),
        pltpu.VMEM((TOK_PER_W,), jnp.float32),
        pltpu.SemaphoreType.DMA,
    ],
)
def _sc_gather(idx_hbm, packed_hbm, out_hbm, idx_v, val_v, sem):
    wid = lax.axis_index("s") * NC + lax.axis_index("c")
    base = wid * TOK_PER_W
    pltpu.sync_copy(idx_hbm.at[pl.ds(base, TOK_PER_W)], idx_v)
    copies = []
    for j in range(N_CHUNKS):
        copies.append(
            pltpu.async_copy(
                packed_hbm.at[idx_v.at[pl.ds(j * CHUNK, CHUNK)]],
                val_v.at[pl.ds(j * CHUNK, CHUNK)],
                sem,
            )
        )
    for cp in copies:
        cp.wait()
    pltpu.sync_copy(val_v, out_hbm.at[pl.ds(base, TOK_PER_W)])


TOK_BLK = 25600
GRID = N_TOK // TOK_BLK


def _tc_body(packed_ref, w_ref, out_ref):
    p = packed_ref[...].astype(jnp.int32)
    x0 = (p & 15) - 5
    x1 = ((p >> 4) & 15) - 5
    x2 = ((p >> 8) & 15) - 5
    x3 = (p >> 12) - 5
    x = jnp.concatenate([x0, x1, x2, x3], axis=0).astype(jnp.bfloat16)
    xx = x * x
    xxx = xx * x
    ones = jnp.ones((1, TOK_BLK), jnp.bfloat16)
    pw = jnp.concatenate([ones, x, xx, xxx], axis=0)
    out_ref[...] = lax.dot_general(
        pw, w_ref[...], (((0,), (0,)), ((), ())),
        preferred_element_type=jnp.float32,
    )


def _tc_dense(packed_plane, w13):
    return pl.pallas_call(
        _tc_body,
        grid=(GRID,),
        in_specs=[
            pl.BlockSpec((1, TOK_BLK), lambda i: (0, i)),
            pl.BlockSpec((3 * N_TURNS + 1, OUT_DIM), lambda i: (0, 0)),
        ],
        out_specs=pl.BlockSpec((TOK_BLK, OUT_DIM), lambda i: (i, 0)),
        out_shape=jax.ShapeDtypeStruct((N_TOK, OUT_DIM), jnp.float32),
    )(packed_plane, w13)


def kernel(token_ids, turns, poly_coeffs):
    idx1d = token_ids.T.reshape(N_TOK)
    packed = _packed_table(turns)
    plane = _sc_gather(idx1d, packed)
    w12 = poly_coeffs[:, 1:, :].transpose(1, 0, 2).reshape(3 * N_TURNS, OUT_DIM)
    bias = jnp.sum(poly_coeffs[:, 0, :], axis=0).reshape(1, OUT_DIM)
    w13 = jnp.concatenate([bias, w12], axis=0).astype(jnp.bfloat16)
    out2d = _tc_dense(plane.reshape(1, N_TOK), w13)
    return out2d.reshape(S, B, OUT_DIM).transpose(1, 0, 2)


# speedup vs baseline: 1.4646x; 1.0476x over previous
import functools

import jax
import jax.numpy as jnp
import numpy as np
from jax import lax
from jax.experimental import layout as jex_layout
from jax.experimental import pallas as pl
from jax.experimental.pallas import tpu as pltpu
from jax.experimental.pallas import tpu_sc as plsc

B = 4096
S = 50
N_TOK = B * S
VOCAB = 1000000
N_TURNS = 4
OUT_DIM = 128

NC = 2
NS = 16
NW = NC * NS
TOK_PER_W = N_TOK // NW
CHUNK = 128
N_CHUNKS = TOK_PER_W // CHUNK

PANELS = VOCAB // 128
MAIN = PANELS * 128
TAIL = VOCAB - MAIN
PACK_OFF = np.float32(21845.0)

PB = 1024
PACK_GRID = -(-PANELS // PB)
MB_SUB = 128

_pack_m = np.kron(np.eye(MB_SUB, dtype=np.float32),
                  np.array([[1.0, 16.0, 256.0, 4096.0]], np.float32))


def _pack_body(m_ref, x_ref, out_ref):
    m = m_ref[...]
    for k in range(PB // MB_SUB):
        xb = x_ref[pl.ds(k * 4 * MB_SUB, 4 * MB_SUB), :].astype(jnp.bfloat16)
        out_ref[pl.ds(k * MB_SUB, MB_SUB), :] = lax.dot_general(
            m, xb, (((1,), (0,)), ((), ())),
            preferred_element_type=jnp.float32,
        ) + PACK_OFF


def _pack_main(panels):
    return pl.pallas_call(
        _pack_body,
        grid=(PACK_GRID,),
        in_specs=[
            pl.BlockSpec((MB_SUB, 4 * MB_SUB), lambda i: (0, 0)),
            pl.BlockSpec((4 * PB, 128), lambda i: (i, 0)),
        ],
        out_specs=pl.BlockSpec((PB, 128), lambda i: (i, 0)),
        out_shape=jax.ShapeDtypeStruct((PANELS, 128), jnp.float32),
    )(jnp.asarray(_pack_m, jnp.bfloat16), panels)


def _packed_table(turns):
    view3 = turns.T[:, :MAIN].reshape(N_TURNS, PANELS, 128).transpose(1, 0, 2)
    view3 = jex_layout.with_layout_constraint(
        view3, jex_layout.Layout((0, 1, 2), tiling=((4, 128),))
    )
    panels = view3.reshape(PANELS * N_TURNS, 128)
    packed_main = _pack_main(panels).reshape(MAIN)
    radix_w = jnp.array([1.0, 16.0, 256.0, 4096.0], jnp.float32)
    packed_tail = turns[MAIN:] @ radix_w + PACK_OFF
    return jnp.concatenate([packed_main, packed_tail])


_sc_mesh = plsc.VectorSubcoreMesh(core_axis_name="c", subcore_axis_name="s")


@functools.partial(
    pl.kernel,
    mesh=_sc_mesh,
    out_type=jax.ShapeDtypeStruct((N_TOK,), jnp.float32),
    scratch_types=[
        pltpu.VMEM((TOK_PER_W,), jnp.int32),
        pltpu.VMEM((TOK_PER_W,), jnp.float32),
        pltpu.SemaphoreType.DMA,
    ],
)
def _sc_gather(idx_hbm, packed_hbm, out_hbm, idx_v, val_v, sem):
    wid = lax.axis_index("s") * NC + lax.axis_index("c")
    base = wid * TOK_PER_W
    pltpu.sync_copy(idx_hbm.at[pl.ds(base, TOK_PER_W)], idx_v)
    copies = []
    for j in range(N_CHUNKS):
        copies.append(
            pltpu.async_copy(
                packed_hbm.at[idx_v.at[pl.ds(j * CHUNK, CHUNK)]],
                val_v.at[pl.ds(j * CHUNK, CHUNK)],
                sem,
            )
        )
    for cp in copies:
        cp.wait()
    pltpu.sync_copy(val_v, out_hbm.at[pl.ds(base, TOK_PER_W)])


TOK_BLK = 25600
GRID = N_TOK // TOK_BLK


def _tc_body(packed_ref, w_ref, out_ref):
    p = packed_ref[...].astype(jnp.int32)
    x0 = (p & 15) - 5
    x1 = ((p >> 4) & 15) - 5
    x2 = ((p >> 8) & 15) - 5
    x3 = (p >> 12) - 5
    x = jnp.concatenate([x0, x1, x2, x3], axis=0).astype(jnp.bfloat16)
    xx = x * x
    xxx = xx * x
    ones = jnp.ones((1, TOK_BLK), jnp.bfloat16)
    pw = jnp.concatenate([ones, x, xx, xxx], axis=0)
    out_ref[...] = lax.dot_general(
        pw, w_ref[...], (((0,), (0,)), ((), ())),
        preferred_element_type=jnp.float32,
    )


def _tc_dense(packed_plane, w13):
    return pl.pallas_call(
        _tc_body,
        grid=(GRID,),
        in_specs=[
            pl.BlockSpec((1, TOK_BLK), lambda i: (0, i)),
            pl.BlockSpec((3 * N_TURNS + 1, OUT_DIM), lambda i: (0, 0)),
        ],
        out_specs=pl.BlockSpec((TOK_BLK, OUT_DIM), lambda i: (i, 0)),
        out_shape=jax.ShapeDtypeStruct((N_TOK, OUT_DIM), jnp.float32),
    )(packed_plane, w13)


def kernel(token_ids, turns, poly_coeffs):
    idx1d = token_ids.T.reshape(N_TOK)
    packed = _packed_table(turns)
    plane = _sc_gather(idx1d, packed)
    w12 = poly_coeffs[:, 1:, :].transpose(1, 0, 2).reshape(3 * N_TURNS, OUT_DIM)
    bias = jnp.sum(poly_coeffs[:, 0, :], axis=0).reshape(1, OUT_DIM)
    w13 = jnp.concatenate([bias, w12], axis=0).astype(jnp.bfloat16)
    out2d = _tc_dense(plane.reshape(1, N_TOK), w13)
    return out2d.reshape(S, B, OUT_DIM).transpose(1, 0, 2)


# speedup vs baseline: 1.4964x; 1.0217x over previous
import functools

import jax
import jax.numpy as jnp
import numpy as np
from jax import lax
from jax.experimental import layout as jex_layout
from jax.experimental import pallas as pl
from jax.experimental.pallas import tpu as pltpu
from jax.experimental.pallas import tpu_sc as plsc

B = 4096
S = 50
N_TOK = B * S
VOCAB = 1000000
N_TURNS = 4
OUT_DIM = 128

NC = 2
NS = 16
NW = NC * NS
TOK_PER_W = N_TOK // NW
CHUNK = 128
N_CHUNKS = TOK_PER_W // CHUNK

PANELS = VOCAB // 128
MAIN = PANELS * 128
TAIL = VOCAB - MAIN
PACK_OFF = np.float32(21845.0)

PB = 2048
PACK_GRID = -(-PANELS // PB)
MB_SUB = 128

_pack_m = np.kron(np.eye(MB_SUB, dtype=np.float32),
                  np.array([[1.0, 16.0, 256.0, 4096.0]], np.float32))


def _pack_body(m_ref, x_ref, out_ref):
    m = m_ref[...]
    for k in range(PB // MB_SUB):
        xb = x_ref[pl.ds(k * 4 * MB_SUB, 4 * MB_SUB), :].astype(jnp.bfloat16)
        out_ref[pl.ds(k * MB_SUB, MB_SUB), :] = lax.dot_general(
            m, xb, (((1,), (0,)), ((), ())),
            preferred_element_type=jnp.float32,
        ) + PACK_OFF


def _pack_main(panels):
    return pl.pallas_call(
        _pack_body,
        grid=(PACK_GRID,),
        in_specs=[
            pl.BlockSpec((MB_SUB, 4 * MB_SUB), lambda i: (0, 0)),
            pl.BlockSpec((4 * PB, 128), lambda i: (i, 0)),
        ],
        out_specs=pl.BlockSpec((PB, 128), lambda i: (i, 0)),
        out_shape=jax.ShapeDtypeStruct((PANELS, 128), jnp.float32),
    )(jnp.asarray(_pack_m, jnp.bfloat16), panels)


def _packed_table(turns):
    view3 = turns.T[:, :MAIN].reshape(N_TURNS, PANELS, 128).transpose(1, 0, 2)
    view3 = jex_layout.with_layout_constraint(
        view3, jex_layout.Layout((0, 1, 2), tiling=((4, 128),))
    )
    panels = view3.reshape(PANELS * N_TURNS, 128)
    packed_main = _pack_main(panels).reshape(MAIN)
    radix_w = jnp.array([1.0, 16.0, 256.0, 4096.0], jnp.float32)
    packed_tail = turns[MAIN:] @ radix_w + PACK_OFF
    return jnp.concatenate([packed_main, packed_tail])


_sc_mesh = plsc.VectorSubcoreMesh(core_axis_name="c", subcore_axis_name="s")


@functools.partial(
    pl.kernel,
    mesh=_sc_mesh,
    out_type=jax.ShapeDtypeStruct((N_TOK,), jnp.float32),
    scratch_types=[
        pltpu.VMEM((TOK_PER_W,), jnp.int32),
        pltpu.VMEM((TOK_PER_W,), jnp.float32),
        pltpu.SemaphoreType.DMA,
    ],
)
def _sc_gather(idx_hbm, packed_hbm, out_hbm, idx_v, val_v, sem):
    wid = lax.axis_index("s") * NC + lax.axis_index("c")
    base = wid * TOK_PER_W
    pltpu.sync_copy(idx_hbm.at[pl.ds(base, TOK_PER_W)], idx_v)
    copies = []
    for j in range(N_CHUNKS):
        copies.append(
            pltpu.async_copy(
                packed_hbm.at[idx_v.at[pl.ds(j * CHUNK, CHUNK)]],
                val_v.at[pl.ds(j * CHUNK, CHUNK)],
                sem,
            )
        )
    for cp in copies:
        cp.wait()
    pltpu.sync_copy(val_v, out_hbm.at[pl.ds(base, TOK_PER_W)])


TOK_BLK = 25600
GRID = N_TOK // TOK_BLK


def _tc_body(packed_ref, w_ref, out_ref):
    p = packed_ref[...].astype(jnp.int32)
    x0 = (p & 15) - 5
    x1 = ((p >> 4) & 15) - 5
    x2 = ((p >> 8) & 15) - 5
    x3 = (p >> 12) - 5
    x = jnp.concatenate([x0, x1, x2, x3], axis=0).astype(jnp.bfloat16)
    xx = x * x
    xxx = xx * x
    ones = jnp.ones((1, TOK_BLK), jnp.bfloat16)
    pw = jnp.concatenate([ones, x, xx, xxx], axis=0)
    out_ref[...] = lax.dot_general(
        pw, w_ref[...], (((0,), (0,)), ((), ())),
        preferred_element_type=jnp.float32,
    )


def _tc_dense(packed_plane, w13):
    return pl.pallas_call(
        _tc_body,
        grid=(GRID,),
        in_specs=[
            pl.BlockSpec((1, TOK_BLK), lambda i: (0, i)),
            pl.BlockSpec((3 * N_TURNS + 1, OUT_DIM), lambda i: (0, 0)),
        ],
        out_specs=pl.BlockSpec((TOK_BLK, OUT_DIM), lambda i: (i, 0)),
        out_shape=jax.ShapeDtypeStruct((N_TOK, OUT_DIM), jnp.float32),
    )(packed_plane, w13)


def kernel(token_ids, turns, poly_coeffs):
    idx1d = token_ids.T.reshape(N_TOK)
    packed = _packed_table(turns)
    plane = _sc_gather(idx1d, packed)
    w12 = poly_coeffs[:, 1:, :].transpose(1, 0, 2).reshape(3 * N_TURNS, OUT_DIM)
    bias = jnp.sum(poly_coeffs[:, 0, :], axis=0).reshape(1, OUT_DIM)
    w13 = jnp.concatenate([bias, w12], axis=0).astype(jnp.bfloat16)
    out2d = _tc_dense(plane.reshape(1, N_TOK), w13)
    return out2d.reshape(S, B, OUT_DIM).transpose(1, 0, 2)
